# baseline (speedup 1.0000x reference)
import jax
import jax.numpy as jnp
from jax import lax
from jax.experimental import pallas as pl
from jax.experimental.pallas import tpu as pltpu

_W = 8
_ALPHA = 0.5
_TB = 128
_R = 2


def _stat_mem_kernel(z_ref, v_ref, h_ref, yb_ref, vb_ref, sc_ref, mem_ref):
    tb = pl.program_id(1)
    TB = z_ref.shape[1]
    W = _W

    @pl.when(tb == 0)
    def _():
        yb_ref[:, :W] = jnp.zeros_like(yb_ref[:, :W])
        vb_ref[:, :W] = jnp.zeros_like(vb_ref[:, :W])
        mem_ref[...] = jnp.zeros_like(mem_ref)

    vc = v_ref[...]
    vb_ref[:, W:] = vc
    yb_ref[:, W:] = vc * z_ref[...]

    ve = vb_ref[...]

    @pl.when(tb == 0)
    def _():
        r = lax.broadcasted_iota(jnp.int32, (1, TB, 1, 1), 1).astype(jnp.float32)
        lwin = jnp.minimum(r + 1.0, jnp.float32(W))
        num = None
        den = None
        norm = None
        for k in range(W):
            wk = jnp.where(r >= jnp.float32(k), lwin - jnp.float32(1 + 2 * k), 0.0)
            awk = jnp.abs(wk)
            yk = yb_ref[:, W - k:W - k + TB]
            vk = ve[:, W - k:W - k + TB]
            if num is None:
                num, den, norm = wk * yk, awk * vk, awk
            else:
                num = num + wk * yk
                den = den + awk * vk
                norm = norm + awk
        clamp = jnp.maximum(den, 1e-6 * jnp.maximum(norm, 1e-6))
        scale = (_ALPHA * vc) / clamp
        sc_ref[...] = scale
        h_ref[...] = scale * num

    @pl.when(tb > 0)
    def _():
        den = (7.0 * (ve[:, 8:8 + TB] + ve[:, 1:1 + TB])
               + 5.0 * (ve[:, 7:7 + TB] + ve[:, 2:2 + TB])
               + 3.0 * (ve[:, 6:6 + TB] + ve[:, 3:3 + TB])
               + (ve[:, 5:5 + TB] + ve[:, 4:4 + TB]))
        clamp = jnp.maximum(den, jnp.float32(32e-6))
        sc_ref[...] = (_ALPHA * vc) / clamp

        def _chunk(i, mem):
            b = i * _R
            y0 = yb_ref[:, pl.ds(b + 8, _R)]
            y1 = yb_ref[:, pl.ds(b + 7, _R)]
            y2 = yb_ref[:, pl.ds(b + 6, _R)]
            y3 = yb_ref[:, pl.ds(b + 5, _R)]
            y4 = yb_ref[:, pl.ds(b + 4, _R)]
            y5 = yb_ref[:, pl.ds(b + 3, _R)]
            y6 = yb_ref[:, pl.ds(b + 2, _R)]
            y7 = yb_ref[:, pl.ds(b + 1, _R)]
            num = (7.0 * (y0 - y7) + 5.0 * (y1 - y6)
                   + 3.0 * (y2 - y5) + (y3 - y4))
            bv = sc_ref[:, pl.ds(b, _R)] * num
            for j in range(_R):
                vt = vb_ref[:, b + W + j]
                at = 1.0 - _ALPHA * vt
                mem = at * mem + bv[:, j]
                h_ref[:, b + j] = mem
            return mem

        mem_ref[...] = lax.fori_loop(0, TB // _R, _chunk, mem_ref[...])

    @pl.when(tb == 0)
    def _():
        def _row(i, mem):
            vt = vb_ref[:, W + i]
            at = 1.0 - _ALPHA * vt
            h = at * mem + h_ref[:, i]
            h_ref[:, i] = h
            return h

        mem_ref[...] = lax.fori_loop(0, TB, _row, mem_ref[...], unroll=8)

    yb_ref[:, :W] = yb_ref[:, TB:TB + W]
    vb_ref[:, :W] = vb_ref[:, TB:TB + W]


def kernel(z, valid_mask):
    B, T, M, D = z.shape
    TB = _TB
    NT = T // TB
    BB = B // 2
    v4 = valid_mask[..., None]

    h = pl.pallas_call(
        _stat_mem_kernel,
        grid=(2, NT),
        in_specs=[
            pl.BlockSpec((BB, TB, M, D), lambda c, t: (c, t, 0, 0)),
            pl.BlockSpec((BB, TB, M, 1), lambda c, t: (c, t, 0, 0)),
        ],
        out_specs=pl.BlockSpec((BB, TB, M, D), lambda c, t: (c, t, 0, 0)),
        out_shape=jax.ShapeDtypeStruct((B, T, M, D), z.dtype),
        scratch_shapes=[
            pltpu.VMEM((BB, TB + _W, M, D), jnp.float32),
            pltpu.VMEM((BB, TB + _W, M, 1), jnp.float32),
            pltpu.VMEM((BB, TB, M, 1), jnp.float32),
            pltpu.VMEM((BB, M, D), jnp.float32),
        ],
        compiler_params=pltpu.CompilerParams(
            dimension_semantics=("parallel", "arbitrary"),
        ),
        name="stat_mem",
    )(z, v4)
    return h, h[:, -1]


# speedup vs baseline: 1.1533x; 1.1533x over previous
import jax
import jax.numpy as jnp
from jax import lax
from jax.experimental import pallas as pl
from jax.experimental.pallas import tpu as pltpu

_W = 8
_ALPHA = 0.5
_TB = 128
_R = 4


def _stat_mem_kernel(z_ref, v_ref, h_ref, yb_ref, vb_ref, sc_ref, mem_ref):
    tb = pl.program_id(1)
    TB = z_ref.shape[1]
    W = _W

    @pl.when(tb == 0)
    def _():
        yb_ref[:, :W] = jnp.zeros_like(yb_ref[:, :W])
        vb_ref[:, :W] = jnp.zeros_like(vb_ref[:, :W])
        mem_ref[...] = jnp.zeros_like(mem_ref)

    vc = v_ref[...]
    vb_ref[:, W:] = vc
    yb_ref[:, W:] = vc * z_ref[...]

    ve = vb_ref[...]

    @pl.when(tb == 0)
    def _():
        r = lax.broadcasted_iota(jnp.int32, (1, TB, 1, 1), 1).astype(jnp.float32)
        lwin = jnp.minimum(r + 1.0, jnp.float32(W))
        num = None
        den = None
        norm = None
        for k in range(W):
            wk = jnp.where(r >= jnp.float32(k), lwin - jnp.float32(1 + 2 * k), 0.0)
            awk = jnp.abs(wk)
            yk = yb_ref[:, W - k:W - k + TB]
            vk = ve[:, W - k:W - k + TB]
            if num is None:
                num, den, norm = wk * yk, awk * vk, awk
            else:
                num = num + wk * yk
                den = den + awk * vk
                norm = norm + awk
        clamp = jnp.maximum(den, 1e-6 * jnp.maximum(norm, 1e-6))
        scale = (_ALPHA * vc) / clamp
        sc_ref[...] = scale
        h_ref[...] = scale * num

    @pl.when(tb > 0)
    def _():
        den = (7.0 * (ve[:, 8:8 + TB] + ve[:, 1:1 + TB])
               + 5.0 * (ve[:, 7:7 + TB] + ve[:, 2:2 + TB])
               + 3.0 * (ve[:, 6:6 + TB] + ve[:, 3:3 + TB])
               + (ve[:, 5:5 + TB] + ve[:, 4:4 + TB]))
        clamp = jnp.maximum(den, jnp.float32(32e-6))
        sc_ref[...] = (_ALPHA * vc) / clamp

        def _chunk(i, _):
            b = i * _R
            y0 = yb_ref[:, pl.ds(b + 8, _R)]
            y1 = yb_ref[:, pl.ds(b + 7, _R)]
            y2 = yb_ref[:, pl.ds(b + 6, _R)]
            y3 = yb_ref[:, pl.ds(b + 5, _R)]
            y4 = yb_ref[:, pl.ds(b + 4, _R)]
            y5 = yb_ref[:, pl.ds(b + 3, _R)]
            y6 = yb_ref[:, pl.ds(b + 2, _R)]
            y7 = yb_ref[:, pl.ds(b + 1, _R)]
            num = (7.0 * (y0 - y7) + 5.0 * (y1 - y6)
                   + 3.0 * (y2 - y5) + (y3 - y4))
            h_ref[:, pl.ds(b, _R)] = sc_ref[:, pl.ds(b, _R)] * num
            return 0

        lax.fori_loop(0, TB // _R, _chunk, 0)

    def _row(i, mem):
        vt = vb_ref[:, W + i]
        at = 1.0 - _ALPHA * vt
        h = at * mem + h_ref[:, i]
        h_ref[:, i] = h
        return h

    mem_ref[...] = lax.fori_loop(0, TB, _row, mem_ref[...], unroll=8)

    yb_ref[:, :W] = yb_ref[:, TB:TB + W]
    vb_ref[:, :W] = vb_ref[:, TB:TB + W]


def kernel(z, valid_mask):
    B, T, M, D = z.shape
    TB = _TB
    NT = T // TB
    BB = B // 2
    v4 = valid_mask[..., None]

    h = pl.pallas_call(
        _stat_mem_kernel,
        grid=(2, NT),
        in_specs=[
            pl.BlockSpec((BB, TB, M, D), lambda c, t: (c, t, 0, 0)),
            pl.BlockSpec((BB, TB, M, 1), lambda c, t: (c, t, 0, 0)),
        ],
        out_specs=pl.BlockSpec((BB, TB, M, D), lambda c, t: (c, t, 0, 0)),
        out_shape=jax.ShapeDtypeStruct((B, T, M, D), z.dtype),
        scratch_shapes=[
            pltpu.VMEM((BB, TB + _W, M, D), jnp.float32),
            pltpu.VMEM((BB, TB + _W, M, 1), jnp.float32),
            pltpu.VMEM((BB, TB, M, 1), jnp.float32),
            pltpu.VMEM((BB, M, D), jnp.float32),
        ],
        compiler_params=pltpu.CompilerParams(
            dimension_semantics=("parallel", "arbitrary"),
        ),
        name="stat_mem",
    )(z, v4)
    return h, h[:, -1]


# speedup vs baseline: 1.2597x; 1.0923x over previous
import jax
import jax.numpy as jnp
from jax import lax
from jax.experimental import pallas as pl
from jax.experimental.pallas import tpu as pltpu

_W = 8
_ALPHA = 0.5
_TB = 128
_R = 4


def _stat_mem_kernel(z_ref, v_ref, h_ref, yb_ref, vb_ref, sc_ref, b2_ref, mem_ref):
    tb = pl.program_id(1)
    TB = z_ref.shape[1]
    W = _W

    @pl.when(tb == 0)
    def _():
        yb_ref[:, :W] = jnp.zeros_like(yb_ref[:, :W])
        vb_ref[:, :W] = jnp.zeros_like(vb_ref[:, :W])
        mem_ref[...] = jnp.zeros_like(mem_ref)

    vc = v_ref[...]
    vb_ref[:, W:] = vc
    yb_ref[:, W:] = vc * z_ref[...]

    ve = vb_ref[...]

    @pl.when(tb == 0)
    def _():
        r = lax.broadcasted_iota(jnp.int32, (1, TB, 1, 1), 1).astype(jnp.float32)
        lwin = jnp.minimum(r + 1.0, jnp.float32(W))
        num = None
        den = None
        norm = None
        for k in range(W):
            wk = jnp.where(r >= jnp.float32(k), lwin - jnp.float32(1 + 2 * k), 0.0)
            awk = jnp.abs(wk)
            yk = yb_ref[:, W - k:W - k + TB]
            vk = ve[:, W - k:W - k + TB]
            if num is None:
                num, den, norm = wk * yk, awk * vk, awk
            else:
                num = num + wk * yk
                den = den + awk * vk
                norm = norm + awk
        clamp = jnp.maximum(den, 1e-6 * jnp.maximum(norm, 1e-6))
        scale = (_ALPHA * vc) / clamp
        sc_ref[...] = scale
        h_ref[...] = scale * num

    @pl.when(tb > 0)
    def _():
        den = (7.0 * (ve[:, 8:8 + TB] + ve[:, 1:1 + TB])
               + 5.0 * (ve[:, 7:7 + TB] + ve[:, 2:2 + TB])
               + 3.0 * (ve[:, 6:6 + TB] + ve[:, 3:3 + TB])
               + (ve[:, 5:5 + TB] + ve[:, 4:4 + TB]))
        clamp = jnp.maximum(den, jnp.float32(32e-6))
        sc_ref[...] = (_ALPHA * vc) / clamp

        def _chunk(i, _):
            b = i * _R
            y0 = yb_ref[:, pl.ds(b + 8, _R)]
            y1 = yb_ref[:, pl.ds(b + 7, _R)]
            y2 = yb_ref[:, pl.ds(b + 6, _R)]
            y3 = yb_ref[:, pl.ds(b + 5, _R)]
            y4 = yb_ref[:, pl.ds(b + 4, _R)]
            y5 = yb_ref[:, pl.ds(b + 3, _R)]
            y6 = yb_ref[:, pl.ds(b + 2, _R)]
            y7 = yb_ref[:, pl.ds(b + 1, _R)]
            num = (7.0 * (y0 - y7) + 5.0 * (y1 - y6)
                   + 3.0 * (y2 - y5) + (y3 - y4))
            h_ref[:, pl.ds(b, _R)] = sc_ref[:, pl.ds(b, _R)] * num
            return 0

        lax.fori_loop(0, TB // _R, _chunk, 0, unroll=2)

    TBH = TB // 2
    av = 1.0 - _ALPHA * ve[:, W:]
    ar = av.reshape(av.shape[0], TBH, 2, av.shape[2], 1)
    a_ev = ar[:, :, 0]
    a_od = ar[:, :, 1]
    sc_ref[:, :TBH] = a_od * a_ev
    sc_ref[:, TBH:] = a_ev
    hv = h_ref[...]
    hr = hv.reshape(hv.shape[0], TBH, 2, hv.shape[2], hv.shape[3])
    b2_ref[...] = a_od * hr[:, :, 0] + hr[:, :, 1]

    def _pair(j, mem):
        aev = sc_ref[:, TBH + j]
        h_ref[:, 2 * j] = aev * mem + h_ref[:, 2 * j]
        m2 = sc_ref[:, j] * mem + b2_ref[:, j]
        h_ref[:, 2 * j + 1] = m2
        return m2

    mem_ref[...] = lax.fori_loop(0, TBH, _pair, mem_ref[...], unroll=4)

    yb_ref[:, :W] = yb_ref[:, TB:TB + W]
    vb_ref[:, :W] = vb_ref[:, TB:TB + W]


def kernel(z, valid_mask):
    B, T, M, D = z.shape
    TB = _TB
    NT = T // TB
    BB = B // 2
    v4 = valid_mask[..., None]

    h = pl.pallas_call(
        _stat_mem_kernel,
        grid=(2, NT),
        in_specs=[
            pl.BlockSpec((BB, TB, M, D), lambda c, t: (c, t, 0, 0)),
            pl.BlockSpec((BB, TB, M, 1), lambda c, t: (c, t, 0, 0)),
        ],
        out_specs=pl.BlockSpec((BB, TB, M, D), lambda c, t: (c, t, 0, 0)),
        out_shape=jax.ShapeDtypeStruct((B, T, M, D), z.dtype),
        scratch_shapes=[
            pltpu.VMEM((BB, TB + _W, M, D), jnp.float32),
            pltpu.VMEM((BB, TB + _W, M, 1), jnp.float32),
            pltpu.VMEM((BB, TB, M, 1), jnp.float32),
            pltpu.VMEM((BB, TB // 2, M, D), jnp.float32),
            pltpu.VMEM((BB, M, D), jnp.float32),
        ],
        compiler_params=pltpu.CompilerParams(
            dimension_semantics=("parallel", "arbitrary"),
        ),
        name="stat_mem",
    )(z, v4)
    return h, h[:, -1]


# speedup vs baseline: 1.4340x; 1.1384x over previous
import jax
import jax.numpy as jnp
from jax import lax
from jax.experimental import pallas as pl
from jax.experimental.pallas import tpu as pltpu

_W = 8
_ALPHA = 0.5
_TB = 128
_R = 4


def _stat_mem_kernel(z_ref, v_ref, h_ref, yb_ref, vb_ref, sc_ref, b2_ref, mem_ref):
    tb = pl.program_id(1)
    TB = z_ref.shape[1]
    W = _W

    @pl.when(tb == 0)
    def _():
        yb_ref[:, :W] = jnp.zeros_like(yb_ref[:, :W])
        vb_ref[:, :W] = jnp.zeros_like(vb_ref[:, :W])
        mem_ref[...] = jnp.zeros_like(mem_ref)

    vc = v_ref[...][..., None]
    vb_ref[:, W:] = vc
    yb_ref[:, W:] = vc * z_ref[...]

    ve = vb_ref[...]

    @pl.when(tb == 0)
    def _():
        r = lax.broadcasted_iota(jnp.int32, (1, TB, 1, 1), 1).astype(jnp.float32)
        lwin = jnp.minimum(r + 1.0, jnp.float32(W))
        num = None
        den = None
        norm = None
        for k in range(W):
            wk = jnp.where(r >= jnp.float32(k), lwin - jnp.float32(1 + 2 * k), 0.0)
            awk = jnp.abs(wk)
            yk = yb_ref[:, W - k:W - k + TB]
            vk = ve[:, W - k:W - k + TB]
            if num is None:
                num, den, norm = wk * yk, awk * vk, awk
            else:
                num = num + wk * yk
                den = den + awk * vk
                norm = norm + awk
        clamp = jnp.maximum(den, 1e-6 * jnp.maximum(norm, 1e-6))
        scale = (_ALPHA * vc) / clamp
        sc_ref[...] = scale
        h_ref[...] = scale * num

    @pl.when(tb > 0)
    def _():
        den = (7.0 * (ve[:, 8:8 + TB] + ve[:, 1:1 + TB])
               + 5.0 * (ve[:, 7:7 + TB] + ve[:, 2:2 + TB])
               + 3.0 * (ve[:, 6:6 + TB] + ve[:, 3:3 + TB])
               + (ve[:, 5:5 + TB] + ve[:, 4:4 + TB]))
        clamp = jnp.maximum(den, jnp.float32(32e-6))
        sc_ref[...] = (_ALPHA * vc) / clamp

        def _chunk(i, _):
            b = i * _R
            y0 = yb_ref[:, pl.ds(b + 8, _R)]
            y1 = yb_ref[:, pl.ds(b + 7, _R)]
            y2 = yb_ref[:, pl.ds(b + 6, _R)]
            y3 = yb_ref[:, pl.ds(b + 5, _R)]
            y4 = yb_ref[:, pl.ds(b + 4, _R)]
            y5 = yb_ref[:, pl.ds(b + 3, _R)]
            y6 = yb_ref[:, pl.ds(b + 2, _R)]
            y7 = yb_ref[:, pl.ds(b + 1, _R)]
            num = (7.0 * (y0 - y7) + 5.0 * (y1 - y6)
                   + 3.0 * (y2 - y5) + (y3 - y4))
            h_ref[:, pl.ds(b, _R)] = sc_ref[:, pl.ds(b, _R)] * num
            return 0

        lax.fori_loop(0, TB // _R, _chunk, 0, unroll=2)

    TBH = TB // 2
    av = 1.0 - _ALPHA * ve[:, W:]
    ar = av.reshape(av.shape[0], TBH, 2, av.shape[2], 1)
    a_ev = ar[:, :, 0]
    a_od = ar[:, :, 1]
    sc_ref[:, :TBH] = a_od * a_ev
    sc_ref[:, TBH:] = a_ev
    hv = h_ref[...]
    hr = hv.reshape(hv.shape[0], TBH, 2, hv.shape[2], hv.shape[3])
    b2_ref[...] = a_od * hr[:, :, 0] + hr[:, :, 1]

    def _pair(j, mem):
        aev = sc_ref[:, TBH + j]
        h_ref[:, 2 * j] = aev * mem + h_ref[:, 2 * j]
        m2 = sc_ref[:, j] * mem + b2_ref[:, j]
        h_ref[:, 2 * j + 1] = m2
        return m2

    mem_ref[...] = lax.fori_loop(0, TBH, _pair, mem_ref[...], unroll=4)

    yb_ref[:, :W] = yb_ref[:, TB:TB + W]
    vb_ref[:, :W] = vb_ref[:, TB:TB + W]


def kernel(z, valid_mask):
    B, T, M, D = z.shape
    TB = _TB
    NT = T // TB
    BB = B // 2

    h = pl.pallas_call(
        _stat_mem_kernel,
        grid=(2, NT),
        in_specs=[
            pl.BlockSpec((BB, TB, M, D), lambda c, t: (c, t, 0, 0)),
            pl.BlockSpec((BB, TB, M), lambda c, t: (c, t, 0)),
        ],
        out_specs=pl.BlockSpec((BB, TB, M, D), lambda c, t: (c, t, 0, 0)),
        out_shape=jax.ShapeDtypeStruct((B, T, M, D), z.dtype),
        scratch_shapes=[
            pltpu.VMEM((BB, TB + _W, M, D), jnp.float32),
            pltpu.VMEM((BB, TB + _W, M, 1), jnp.float32),
            pltpu.VMEM((BB, TB, M, 1), jnp.float32),
            pltpu.VMEM((BB, TB // 2, M, D), jnp.float32),
            pltpu.VMEM((BB, M, D), jnp.float32),
        ],
        compiler_params=pltpu.CompilerParams(
            dimension_semantics=("parallel", "arbitrary"),
        ),
        name="stat_mem",
    )(z, valid_mask)
    return h, h[:, -1]


# speedup vs baseline: 1.4386x; 1.0032x over previous
import jax
import jax.numpy as jnp
from jax import lax
from jax.experimental import pallas as pl
from jax.experimental.pallas import tpu as pltpu

_W = 8
_ALPHA = 0.5
_TB = 128
_R = 4


def _stat_mem_kernel(z_ref, v_ref, h_ref, yb_ref, vb_ref, sc_ref, b2_ref, mem_ref):
    tb = pl.program_id(1)
    TB = z_ref.shape[1]
    W = _W

    @pl.when(tb == 0)
    def _():
        yb_ref[:, :W] = jnp.zeros_like(yb_ref[:, :W])
        vb_ref[:, :W] = jnp.zeros_like(vb_ref[:, :W])
        mem_ref[...] = jnp.zeros_like(mem_ref)

    vc = v_ref[...][..., None]
    vb_ref[:, W:] = vc
    yb_ref[:, W:] = vc * z_ref[...]

    ve = vb_ref[...]

    @pl.when(tb == 0)
    def _():
        r = lax.broadcasted_iota(jnp.int32, (1, TB, 1, 1), 1).astype(jnp.float32)
        lwin = jnp.minimum(r + 1.0, jnp.float32(W))
        num = None
        den = None
        norm = None
        for k in range(W):
            wk = jnp.where(r >= jnp.float32(k), lwin - jnp.float32(1 + 2 * k), 0.0)
            awk = jnp.abs(wk)
            yk = yb_ref[:, W - k:W - k + TB]
            vk = ve[:, W - k:W - k + TB]
            if num is None:
                num, den, norm = wk * yk, awk * vk, awk
            else:
                num = num + wk * yk
                den = den + awk * vk
                norm = norm + awk
        clamp = jnp.maximum(den, 1e-6 * jnp.maximum(norm, 1e-6))
        scale = (_ALPHA * vc) / clamp
        sc_ref[...] = scale
        h_ref[...] = scale * num

    @pl.when(tb > 0)
    def _():
        den = (7.0 * (ve[:, 8:8 + TB] + ve[:, 1:1 + TB])
               + 5.0 * (ve[:, 7:7 + TB] + ve[:, 2:2 + TB])
               + 3.0 * (ve[:, 6:6 + TB] + ve[:, 3:3 + TB])
               + (ve[:, 5:5 + TB] + ve[:, 4:4 + TB]))
        clamp = jnp.maximum(den, jnp.float32(32e-6))
        sc_ref[...] = (_ALPHA * vc) / clamp

        nb = z_ref.shape[0]

        def _chunk(i, _):
            b = i * _R
            for bi in range(nb):
                ya = yb_ref[bi, pl.ds(b + 1, _R + 7)]
                num = (7.0 * (ya[7:7 + _R] - ya[0:_R])
                       + 5.0 * (ya[6:6 + _R] - ya[1:1 + _R])
                       + 3.0 * (ya[5:5 + _R] - ya[2:2 + _R])
                       + (ya[4:4 + _R] - ya[3:3 + _R]))
                h_ref[bi, pl.ds(b, _R)] = sc_ref[bi, pl.ds(b, _R)] * num
            return 0

        lax.fori_loop(0, TB // _R, _chunk, 0, unroll=2)

    TBH = TB // 2
    av = 1.0 - _ALPHA * ve[:, W:]
    ar = av.reshape(av.shape[0], TBH, 2, av.shape[2], 1)
    a_ev = ar[:, :, 0]
    a_od = ar[:, :, 1]
    sc_ref[:, :TBH] = a_od * a_ev
    sc_ref[:, TBH:] = a_ev
    hv = h_ref[...]
    hr = hv.reshape(hv.shape[0], TBH, 2, hv.shape[2], hv.shape[3])
    b2_ref[...] = a_od * hr[:, :, 0] + hr[:, :, 1]

    def _pair(j, mem):
        aev = sc_ref[:, TBH + j]
        h_ref[:, 2 * j] = aev * mem + h_ref[:, 2 * j]
        m2 = sc_ref[:, j] * mem + b2_ref[:, j]
        h_ref[:, 2 * j + 1] = m2
        return m2

    mem_ref[...] = lax.fori_loop(0, TBH, _pair, mem_ref[...], unroll=4)

    yb_ref[:, :W] = yb_ref[:, TB:TB + W]
    vb_ref[:, :W] = vb_ref[:, TB:TB + W]


def kernel(z, valid_mask):
    B, T, M, D = z.shape
    TB = _TB
    NT = T // TB
    BB = B // 2

    h = pl.pallas_call(
        _stat_mem_kernel,
        grid=(2, NT),
        in_specs=[
            pl.BlockSpec((BB, TB, M, D), lambda c, t: (c, t, 0, 0)),
            pl.BlockSpec((BB, TB, M), lambda c, t: (c, t, 0)),
        ],
        out_specs=pl.BlockSpec((BB, TB, M, D), lambda c, t: (c, t, 0, 0)),
        out_shape=jax.ShapeDtypeStruct((B, T, M, D), z.dtype),
        scratch_shapes=[
            pltpu.VMEM((BB, TB + _W, M, D), jnp.float32),
            pltpu.VMEM((BB, TB + _W, M, 1), jnp.float32),
            pltpu.VMEM((BB, TB, M, 1), jnp.float32),
            pltpu.VMEM((BB, TB // 2, M, D), jnp.float32),
            pltpu.VMEM((BB, M, D), jnp.float32),
        ],
        compiler_params=pltpu.CompilerParams(
            dimension_semantics=("parallel", "arbitrary"),
        ),
        name="stat_mem",
    )(z, valid_mask)
    return h, h[:, -1]
